# baseline (speedup 1.0000x reference)
import jax
import jax.numpy as jnp
from jax.experimental import pallas as pl
from jax.experimental.pallas import tpu as pltpu

B = 16384
IN_DIM = 100
F_PAD = 128
HIDDEN = 64
GRID = 16
X_MIN, X_MAX = -3.0, 3.0
INV_STEP = (GRID - 1) / (X_MAX - X_MIN)
BB = 1024
Q2_LO = 7
OUT_COLS = 128


def _a_piece(p, q):
    if q == 0:
        return jnp.maximum(1.0 - p, 0.0)
    if q == 1:
        return jnp.minimum(p, jnp.maximum(2.0 - p, 0.0))
    if q == GRID - 2:
        return jnp.minimum(jnp.maximum(p - (GRID - 3.0), 0.0), (GRID - 1.0) - p)
    if q == GRID - 1:
        return jnp.maximum(p - (GRID - 2.0), 0.0)
    return jnp.maximum(1.0 - jnp.abs(p - float(q)), 0.0)


def _split_bf16(a):
    au = jax.lax.bitcast_convert_type(a, jnp.uint32)
    ah = jax.lax.bitcast_convert_type(au & jnp.uint32(0xFFFF0000), jnp.float32)
    return ah, a - ah


def _round_bf16(a):
    au = jax.lax.bitcast_convert_type(a, jnp.uint32)
    rounded = (au + jnp.uint32(0x7FFF) + ((au >> 16) & jnp.uint32(1))) & jnp.uint32(
        0xFFFF0000
    )
    return jax.lax.bitcast_convert_type(rounded, jnp.float32)


def _kan_layer(v, c_cat_ref, bias, q_lo):
    p = (v - X_MIN) * INV_STEP
    a = jnp.concatenate([_a_piece(p, q) for q in range(q_lo, GRID)], axis=1)
    a_hi, a_lo = _split_bf16(a)
    c = c_cat_ref[...]
    r = jnp.dot(
        a_hi.astype(jnp.bfloat16), c, preferred_element_type=jnp.float32
    ) + jnp.dot(a_lo.astype(jnp.bfloat16), c, preferred_element_type=jnp.float32)
    return r[:, :HIDDEN] + r[:, HIDDEN:] + bias


def _fwd_kernel(x_ref, c1cat_ref, c2cat_ref, misc_ref, out_ref):
    misc = misc_ref[...]
    b1 = misc[:, 0:HIDDEN]
    b2 = misc[:, HIDDEN : 2 * HIDDEN]
    wt = misc[:, 2 * HIDDEN : 3 * HIDDEN]
    bout = misc[0, 3 * HIDDEN]
    x = jnp.pad(x_ref[...], ((0, 0), (0, F_PAD - IN_DIM)))
    h = jnp.maximum(_kan_layer(x, c1cat_ref, b1, 0), 0.0)
    h2 = jnp.maximum(_kan_layer(h, c2cat_ref, b2, Q2_LO), 0.0)
    o = jnp.sum(_round_bf16(h2) * wt, axis=1, keepdims=True) + bout
    out_ref[...] = o.reshape(BB // OUT_COLS, OUT_COLS)


@jax.jit
def kernel(x, coeff1, bias1, coeff2, bias2, W, b):
    c1 = jnp.pad(
        jnp.transpose(coeff1, (1, 0, 2)), ((0, 0), (0, F_PAD - IN_DIM), (0, 0))
    ).reshape(GRID * F_PAD, HIDDEN)
    c2 = jnp.transpose(coeff2, (1, 0, 2))[Q2_LO:].reshape(
        (GRID - Q2_LO) * HIDDEN, HIDDEN
    )
    c1_hi, c1_lo = _split_bf16(c1)
    c2_hi, c2_lo = _split_bf16(c2)
    c1_cat = jnp.concatenate([c1_hi, c1_lo], axis=1).astype(jnp.bfloat16)
    c2_cat = jnp.concatenate([c2_hi, c2_lo], axis=1).astype(jnp.bfloat16)
    misc = jnp.concatenate(
        [
            bias1.reshape(1, HIDDEN),
            bias2.reshape(1, HIDDEN),
            _round_bf16(W.reshape(1, HIDDEN)),
            jnp.pad(b.reshape(1, 1), ((0, 0), (0, HIDDEN - 1))),
        ],
        axis=1,
    )

    k1 = GRID * F_PAD
    k2 = (GRID - Q2_LO) * HIDDEN
    n_blocks = B // BB
    out = pl.pallas_call(
        _fwd_kernel,
        grid=(n_blocks,),
        in_specs=[
            pl.BlockSpec((BB, IN_DIM), lambda i: (i, 0)),
            pl.BlockSpec((k1, 2 * HIDDEN), lambda i: (0, 0)),
            pl.BlockSpec((k2, 2 * HIDDEN), lambda i: (0, 0)),
            pl.BlockSpec((1, 4 * HIDDEN), lambda i: (0, 0)),
        ],
        out_specs=pl.BlockSpec((BB // OUT_COLS, OUT_COLS), lambda i: (i, 0)),
        out_shape=jax.ShapeDtypeStruct((B // OUT_COLS, OUT_COLS), jnp.float32),
        compiler_params=pltpu.CompilerParams(
            dimension_semantics=("arbitrary",),
        ),
    )(x, c1_cat, c2_cat, misc)
    return out.reshape(B)


# speedup vs baseline: 1.2743x; 1.2743x over previous
import jax
import jax.numpy as jnp
from jax.experimental import pallas as pl
from jax.experimental.pallas import tpu as pltpu

B = 16384
IN_DIM = 100
F_PAD = 128
HIDDEN = 64
GRID = 16
X_MIN, X_MAX = -3.0, 3.0
INV_STEP = (GRID - 1) / (X_MAX - X_MIN)
BB = 1024
Q2_LO = 7
OUT_COLS = 128


def _a_piece(p, q):
    if q == 0:
        return jnp.maximum(1.0 - p, 0.0)
    if q == 1:
        return jnp.minimum(p, jnp.maximum(2.0 - p, 0.0))
    if q == GRID - 2:
        return jnp.minimum(jnp.maximum(p - (GRID - 3.0), 0.0), (GRID - 1.0) - p)
    if q == GRID - 1:
        return jnp.maximum(p - (GRID - 2.0), 0.0)
    return jnp.maximum(1.0 - jnp.abs(p - float(q)), 0.0)


def _split_bf16(a):
    au = jax.lax.bitcast_convert_type(a, jnp.uint32)
    ah = jax.lax.bitcast_convert_type(au & jnp.uint32(0xFFFF0000), jnp.float32)
    return ah, a - ah


def _round_bf16(a):
    au = jax.lax.bitcast_convert_type(a, jnp.uint32)
    rounded = (au + jnp.uint32(0x7FFF) + ((au >> 16) & jnp.uint32(1))) & jnp.uint32(
        0xFFFF0000
    )
    return jax.lax.bitcast_convert_type(rounded, jnp.float32)


def _kan_layer(v, c_cat_ref, c_hi_ref, bias, q_lo):
    p = (v - X_MIN) * INV_STEP
    a = jnp.concatenate([_a_piece(p, q) for q in range(q_lo, GRID)], axis=1)
    a_hi, a_lo = _split_bf16(a)
    r = jnp.dot(
        a_hi.astype(jnp.bfloat16), c_cat_ref[...],
        preferred_element_type=jnp.float32,
    )
    r_lo = jnp.dot(
        a_lo.astype(jnp.bfloat16), c_hi_ref[...],
        preferred_element_type=jnp.float32,
    )
    return r[:, :HIDDEN] + r[:, HIDDEN:] + r_lo + bias


def _fwd_kernel(x_ref, c1cat_ref, c1hi_ref, c2cat_ref, c2hi_ref, misc_ref, out_ref):
    misc = misc_ref[...]
    b1 = misc[:, 0:HIDDEN]
    b2 = misc[:, HIDDEN : 2 * HIDDEN]
    wt = misc[:, 2 * HIDDEN : 3 * HIDDEN]
    bout = misc[0, 3 * HIDDEN]
    x = jnp.pad(x_ref[...], ((0, 0), (0, F_PAD - IN_DIM)))
    h = jnp.maximum(_kan_layer(x, c1cat_ref, c1hi_ref, b1, 0), 0.0)
    h2 = jnp.maximum(_kan_layer(h, c2cat_ref, c2hi_ref, b2, Q2_LO), 0.0)
    out_ref[...] = jnp.sum(_round_bf16(h2) * wt, axis=1, keepdims=True) + bout


@jax.jit
def kernel(x, coeff1, bias1, coeff2, bias2, W, b):
    c1 = jnp.pad(
        jnp.transpose(coeff1, (1, 0, 2)), ((0, 0), (0, F_PAD - IN_DIM), (0, 0))
    ).reshape(GRID * F_PAD, HIDDEN)
    c2 = jnp.transpose(coeff2, (1, 0, 2))[Q2_LO:].reshape(
        (GRID - Q2_LO) * HIDDEN, HIDDEN
    )
    c1_hi, c1_lo = _split_bf16(c1)
    c2_hi, c2_lo = _split_bf16(c2)
    c1_cat = jnp.concatenate([c1_hi, c1_lo], axis=1).astype(jnp.bfloat16)
    c2_cat = jnp.concatenate([c2_hi, c2_lo], axis=1).astype(jnp.bfloat16)
    c1_hi = c1_hi.astype(jnp.bfloat16)
    c2_hi = c2_hi.astype(jnp.bfloat16)
    misc = jnp.concatenate(
        [
            bias1.reshape(1, HIDDEN),
            bias2.reshape(1, HIDDEN),
            _round_bf16(W.reshape(1, HIDDEN)),
            jnp.pad(b.reshape(1, 1), ((0, 0), (0, HIDDEN - 1))),
        ],
        axis=1,
    )

    k1 = GRID * F_PAD
    k2 = (GRID - Q2_LO) * HIDDEN
    n_blocks = B // BB
    out = pl.pallas_call(
        _fwd_kernel,
        grid=(n_blocks,),
        in_specs=[
            pl.BlockSpec((BB, IN_DIM), lambda i: (i, 0)),
            pl.BlockSpec((k1, 2 * HIDDEN), lambda i: (0, 0)),
            pl.BlockSpec((k1, HIDDEN), lambda i: (0, 0)),
            pl.BlockSpec((k2, 2 * HIDDEN), lambda i: (0, 0)),
            pl.BlockSpec((k2, HIDDEN), lambda i: (0, 0)),
            pl.BlockSpec((1, 4 * HIDDEN), lambda i: (0, 0)),
        ],
        out_specs=pl.BlockSpec((BB, 1), lambda i: (i, 0)),
        out_shape=jax.ShapeDtypeStruct((B, 1), jnp.float32),
        compiler_params=pltpu.CompilerParams(
            dimension_semantics=("arbitrary",),
        ),
    )(x, c1_cat, c1_hi, c2_cat, c2_hi, misc)
    return out[:, 0]
